# baseline (speedup 1.0000x reference)
import functools
import math

import jax
import jax.numpy as jnp
from jax import lax
from jax.experimental import pallas as pl
from jax.experimental.pallas import tpu as pltpu
from jax.experimental.pallas import tpu_sc as plsc

_B = 4096 * 200
_D = 64
_NW = 32
_C = 512
_PER_W = _B // _NW
_ITERS = _PER_W // _C
_SCALE = math.sqrt(_D)

_mesh = plsc.VectorSubcoreMesh(
    core_axis_name="c", subcore_axis_name="s", num_cores=2, num_subcores=16
)


@functools.partial(
    pl.kernel,
    out_type=jax.ShapeDtypeStruct((_B, _D), jnp.float32),
    mesh=_mesh,
    compiler_params=pltpu.CompilerParams(use_tc_tiling_on_sc=False),
    scratch_types=[
        pltpu.VMEM((_C,), jnp.int32),
        pltpu.VMEM((_C, _D), jnp.float32),
        pltpu.SemaphoreType.DMA,
    ],
)
def _embed(tokens_hbm, table_hbm, out_hbm, idx_v, rows_v, sem):
    wid = lax.axis_index("s") * 2 + lax.axis_index("c")
    base = wid * _PER_W

    def chunk(g, carry):
        off = base + g * _C
        pltpu.sync_copy(tokens_hbm.at[pl.ds(off, _C)], idx_v)
        pltpu.async_copy(table_hbm.at[idx_v], rows_v, sem).wait()

        def scale_row(r, carry2):
            for s in range(_D // 16):
                sl = pl.ds(s * 16, 16)
                rows_v[r, sl] = rows_v[r, sl] * _SCALE
            return carry2

        lax.fori_loop(0, _C, scale_row, 0, unroll=8)
        pltpu.sync_copy(rows_v, out_hbm.at[pl.ds(off, _C)])
        return carry

    lax.fori_loop(0, _ITERS, chunk, 0)


def kernel(tokens, table):
    flat = tokens.reshape(_B)
    out = _embed(flat, table)
    return out.reshape(tokens.shape[0], tokens.shape[1], _D)


# speedup vs baseline: 1.0914x; 1.0914x over previous
import functools
import math

import jax
import jax.numpy as jnp
from jax import lax
from jax.experimental import pallas as pl
from jax.experimental.pallas import tpu as pltpu
from jax.experimental.pallas import tpu_sc as plsc

_B = 4096 * 200
_D = 64
_NW = 32
_C = 512
_PER_W = _B // _NW
_ITERS = _PER_W // _C
_SCALE = math.sqrt(_D)

_mesh = plsc.VectorSubcoreMesh(
    core_axis_name="c", subcore_axis_name="s", num_cores=2, num_subcores=16
)


@functools.partial(
    pl.kernel,
    out_type=jax.ShapeDtypeStruct((_B, _D), jnp.float32),
    mesh=_mesh,
    compiler_params=pltpu.CompilerParams(use_tc_tiling_on_sc=False),
    scratch_types=[
        pltpu.VMEM((_PER_W,), jnp.int32),
        pltpu.VMEM((2, _C, _D), jnp.float32),
        pltpu.SemaphoreType.DMA((2,)),
        pltpu.SemaphoreType.DMA((2,)),
    ],
)
def _embed(tokens_hbm, table_hbm, out_hbm, idx_v, rows_v, gsem, osem):
    wid = lax.axis_index("s") * 2 + lax.axis_index("c")
    base = wid * _PER_W

    pltpu.sync_copy(tokens_hbm.at[pl.ds(base, _PER_W)], idx_v)

    def gather_desc(g, b):
        return pltpu.make_async_copy(
            table_hbm.at[idx_v.at[pl.ds(g * _C, _C)]], rows_v.at[b], gsem.at[b]
        )

    def out_desc(g, b):
        return pltpu.make_async_copy(
            rows_v.at[b], out_hbm.at[pl.ds(base + g * _C, _C)], osem.at[b]
        )

    gather_desc(0, 0).start()

    def pair(g2, carry):
        for b in range(2):
            g = 2 * g2 + b
            nb = 1 - b

            @pl.when(g + 1 < _ITERS)
            def _():
                @pl.when(g >= 1)
                def _():
                    out_desc(g - 1, nb).wait()

                gather_desc(g + 1, nb).start()

            gather_desc(g, b).wait()

            def scale_row(r, c2):
                for s in range(_D // 16):
                    sl = pl.ds(s * 16, 16)
                    rows_v[b, r, sl] = rows_v[b, r, sl] * _SCALE
                return c2

            lax.fori_loop(0, _C, scale_row, 0, unroll=8)
            out_desc(g, b).start()
        return carry

    lax.fori_loop(0, _ITERS // 2, pair, 0)

    out_desc(_ITERS - 2, 0).wait()
    out_desc(_ITERS - 1, 1).wait()


def kernel(tokens, table):
    flat = tokens.reshape(_B)
    out = _embed(flat, table)
    return out.reshape(tokens.shape[0], tokens.shape[1], _D)
